# baseline (speedup 1.0000x reference)
import functools

import jax
import jax.numpy as jnp
from jax import lax
from jax.experimental import pallas as pl
from jax.experimental.pallas import tpu as pltpu
from jax.experimental.pallas import tpu_sc as plsc

_F = 26
_V = 100000
_D = 64
_B = 4096

_NC = 2
_NS = 16
_NW = _NC * _NS
_ROWS = _B * _F
_RPW = _ROWS // _NW

_VP = 100096
_NP = _D // 2
_TABP = _F * _NP * _VP

_CR = 128
_NCH = _RPW // _CR

_BB = 512


def _tc_depad_pack(tabT):
    u16 = jnp.uint16
    u32 = jnp.uint32

    def body(in_ref, out_ref):
        for k in range(8):
            a = in_ref[0, 2 * k, :].astype(jnp.bfloat16)
            b = in_ref[0, 2 * k + 1, :].astype(jnp.bfloat16)
            au = lax.bitcast_convert_type(a, u16).astype(u32)
            bu = lax.bitcast_convert_type(b, u16).astype(u32)
            w = au | (bu << 16)
            out_ref[pl.ds(k * _VP, _V)] = lax.bitcast_convert_type(
                w, jnp.int32)

    return pl.pallas_call(
        body,
        grid=(_F, 4),
        in_specs=[pl.BlockSpec((1, 16, _V), lambda f, q: (f, q, 0))],
        out_specs=pl.BlockSpec((8 * _VP,), lambda f, q: (f * 4 + q,)),
        out_shape=jax.ShapeDtypeStruct((_TABP,), jnp.int32),
        compiler_params=pltpu.CompilerParams(
            dimension_semantics=("arbitrary", "arbitrary"),
            vmem_limit_bytes=63 * 1024 * 1024,
        ),
    )(tabT)


def _sc_gather(tab_pk, base_idx):
    mesh = plsc.VectorSubcoreMesh(core_axis_name="c", subcore_axis_name="s")

    @functools.partial(
        pl.kernel,
        mesh=mesh,
        out_type=jax.ShapeDtypeStruct((_ROWS * _NP,), jnp.int32),
        scratch_types=[
            pltpu.VMEM((_RPW,), jnp.int32),
            pltpu.VMEM((_CR * _NP,), jnp.int32),
            pltpu.VMEM((_CR * _NP,), jnp.int32),
            pltpu.VMEM((_CR * _NP,), jnp.int32),
            pltpu.VMEM((_CR * _NP,), jnp.int32),
            pltpu.SemaphoreType.DMA,
            pltpu.SemaphoreType.DMA,
            pltpu.SemaphoreType.DMA,
            pltpu.SemaphoreType.DMA,
        ],
        compiler_params=pltpu.CompilerParams(
            use_tc_tiling_on_sc=False, needs_layout_passes=False),
    )
    def k(tab_hbm, base_hbm, out_hbm, base_v, ei0, ei1, buf0, buf1,
          gs0, gs1, os0, os1):
        wid = lax.axis_index("s") * _NC + lax.axis_index("c")
        rbase = wid * _RPW
        pltpu.sync_copy(base_hbm.at[pl.ds(rbase, _RPW)], base_v)
        eis = (ei0, ei1)
        bufs = (buf0, buf1)
        gsems = (gs0, gs1)
        osems = (os0, os1)

        def gen_idx(c, ei):
            for g in range(_CR // 16):
                j16 = lax.iota(jnp.int32, 16) + (16 * g)
                b16 = base_v[pl.ds(c * _CR + 16 * g, 16)]

                def body_d(d4, _):
                    for u in range(4):
                        d = d4 * 4 + u
                        plsc.store_scatter(
                            ei, [j16 * _NP + d], b16 + d * _VP)
                    return 0

                lax.fori_loop(0, _NP // 4, body_d, 0)

        def issue(c, b):
            pltpu.async_copy(tab_hbm.at[eis[b]], bufs[b], gsems[b])

        gen_idx(0, ei0)
        issue(0, 0)

        def step(i, _):
            for b in range(2):
                c = 2 * i + b
                nb = (b + 1) % 2

                @pl.when(c + 1 < _NCH)
                def _():
                    gen_idx(c + 1, eis[nb])
                    issue(c + 1, nb)

                pltpu.make_async_copy(
                    tab_hbm.at[eis[b]], bufs[b], gsems[b]).wait()

                @pl.when(c >= 2)
                def _():
                    pltpu.make_async_copy(
                        bufs[b], out_hbm.at[pl.ds(0, _CR * _NP)],
                        osems[b]).wait()

                pltpu.async_copy(
                    bufs[b],
                    out_hbm.at[pl.ds((rbase + c * _CR) * _NP, _CR * _NP)],
                    osems[b])
            return 0

        lax.fori_loop(0, _NCH // 2, step, 0)
        for b in range(2):
            pltpu.make_async_copy(
                bufs[b], out_hbm.at[pl.ds(0, _CR * _NP)], osems[b]).wait()

    return k(tab_pk, base_idx)


def _tc_body(dfp, sx, dW0, db0, dW1, db1, dW2, db2,
             cW0, cb0, cW1, cb1, cW2, cb2,
             tW0d, tW0x, tb0, tW1, tb1, tW2, tb2, tW3, tb3,
             fW, fb, out):
    f32 = jnp.float32
    bf = jnp.bfloat16
    mm = lambda a, w: jnp.dot(a.astype(bf), w[...],
                              preferred_element_type=f32)
    x = jnp.maximum(mm(dfp[...], dW0) + db0[...], 0.0)
    x = jnp.maximum(mm(x, dW1) + db1[...], 0.0)
    dxp = jnp.maximum(mm(x, dW2) + db2[...], 0.0)
    x0 = jnp.concatenate([dxp, sx[...].astype(f32)], axis=1)
    xi = x0
    for cW, cb in ((cW0, cb0), (cW1, cb1), (cW2, cb2)):
        xi = x0 * (mm(xi, cW) + cb[...]) + xi
    h = mm(dxp, tW0d) + mm(xi, tW0x) + tb0[...]
    h = jnp.maximum(h, 0.0)
    h = jnp.maximum(mm(h, tW1) + tb1[...], 0.0)
    h = jnp.maximum(mm(h, tW2) + tb2[...], 0.0)
    h = jnp.maximum(mm(h, tW3) + tb3[...], 0.0)
    z = mm(h, fW) + fb[...]
    out[...] = 1.0 / (1.0 + jnp.exp(-z))


def _dense_stack(dfp, sx, weights):
    nb = _B // _BB
    full = lambda s: pl.BlockSpec(s, lambda i: (0, 0))
    in_specs = [
        pl.BlockSpec((_BB, 128), lambda i: (i, 0)),
        pl.BlockSpec((_BB, _F * _D), lambda i: (i, 0)),
    ] + [full(w.shape) for w in weights]
    return pl.pallas_call(
        _tc_body,
        grid=(nb,),
        in_specs=in_specs,
        out_specs=pl.BlockSpec((_BB, 1), lambda i: (i, 0)),
        out_shape=jax.ShapeDtypeStruct((_B, 1), jnp.float32),
        compiler_params=pltpu.CompilerParams(
            dimension_semantics=("arbitrary",),
            vmem_limit_bytes=63 * 1024 * 1024,
        ),
    )(dfp, sx, *weights)


def kernel(dense_features, sparse_features, tables, dense_Ws, dense_bs,
           cross_Ws, cross_bs, top_Ws, top_bs, final_W, final_b):
    f32 = jnp.float32
    tabT = jnp.transpose(tables, (0, 2, 1))
    tab_pk = _tc_depad_pack(tabT)
    base_idx = (sparse_features.astype(jnp.int32)
                + (jnp.arange(_F, dtype=jnp.int32) * (_NP * _VP))[None, :]
                ).reshape(-1)
    words = _sc_gather(tab_pk, base_idx)
    sx = lax.bitcast_convert_type(words, jnp.bfloat16).reshape(_B, _F * _D)

    z = lambda r, c: jnp.zeros((r, c), f32)
    dfp = jnp.pad(dense_features, ((0, 0), (0, 128 - dense_features.shape[1])))
    dW0 = jnp.pad(dense_Ws[0], ((0, 128 - dense_Ws[0].shape[0]), (0, 0)))
    db0 = dense_bs[0].reshape(1, -1)
    dW1 = dense_Ws[1]
    db1 = dense_bs[1].reshape(1, -1)
    dW2 = jnp.pad(dense_Ws[2], ((0, 0), (0, 64)))
    db2 = jnp.pad(dense_bs[2], ((0, 64),)).reshape(1, 128)

    cWs, cbs = [], []
    for W, b in zip(cross_Ws, cross_bs):
        t = jnp.concatenate([W[:64], z(64, W.shape[1]), W[64:]], axis=0)
        Wp = jnp.concatenate([t[:, :64], z(t.shape[0], 64), t[:, 64:]], axis=1)
        cWs.append(Wp)
        cbs.append(jnp.concatenate(
            [b[:64], jnp.zeros((64,), f32), b[64:]]).reshape(1, -1))

    tW0d = jnp.concatenate([top_Ws[0][:64], z(64, 1024)], axis=0)
    tW0x = jnp.concatenate(
        [top_Ws[0][64:128], z(64, 1024), top_Ws[0][128:]], axis=0)
    tb0 = top_bs[0].reshape(1, -1)
    tW1, tb1 = top_Ws[1], top_bs[1].reshape(1, -1)
    tW2, tb2 = top_Ws[2], top_bs[2].reshape(1, -1)
    tW3, tb3 = top_Ws[3], top_bs[3].reshape(1, -1)
    fb = final_b.reshape(1, 1)

    bf = jnp.bfloat16
    weights = [dW0.astype(bf), db0, dW1.astype(bf), db1, dW2.astype(bf), db2,
               cWs[0].astype(bf), cbs[0], cWs[1].astype(bf), cbs[1],
               cWs[2].astype(bf), cbs[2],
               tW0d.astype(bf), tW0x.astype(bf), tb0,
               tW1.astype(bf), tb1, tW2.astype(bf), tb2, tW3.astype(bf), tb3,
               final_W.astype(bf), fb]
    return _dense_stack(dfp, sx, weights)


# speedup vs baseline: 1.7046x; 1.7046x over previous
import functools

import jax
import jax.numpy as jnp
from jax import lax
from jax.experimental import pallas as pl
from jax.experimental.pallas import tpu as pltpu
from jax.experimental.pallas import tpu_sc as plsc

_F = 26
_V = 100000
_VP = 100096
_D = 64
_B = 4096

_NC = 2
_NS = 16
_NW = _NC * _NS
_ROWS = _B * _F
_RPW = _ROWS // _NW

_TAB1 = _F * _D * _VP

_K0_TR = _F * _D // 8
_K0_VW = 4864
_K0_NVW = 20
_K0_TAIL0 = _K0_NVW * _K0_VW
_K0_TAILW = 2688
_K0_TB0 = _K0_TAIL0 + _K0_TAILW
_K0_TBW = _V - _K0_TB0

_HROWS = _ROWS // 2
_HRPW = _HROWS // _NW
_CR = 64
_NCH = _HRPW // _CR

_BB = 512


def _sc_depad(tabT):
    mesh = plsc.VectorSubcoreMesh(core_axis_name="c", subcore_axis_name="s")

    @functools.partial(
        pl.kernel,
        mesh=mesh,
        out_type=jax.ShapeDtypeStruct((_TAB1,), jnp.float32),
        scratch_types=[
            pltpu.VMEM((8, _K0_VW), jnp.float32),
            pltpu.VMEM((8, _K0_VW), jnp.float32),
            pltpu.VMEM((8, _K0_TBW), jnp.float32),
            pltpu.VMEM((8, _K0_TBW), jnp.float32),
            pltpu.SemaphoreType.DMA,
            pltpu.SemaphoreType.DMA,
            pltpu.SemaphoreType.DMA,
            pltpu.SemaphoreType.DMA,
        ],
        compiler_params=pltpu.CompilerParams(
            use_tc_tiling_on_sc=True, needs_layout_passes=False),
    )
    def k(tab_hbm, out_hbm, buf0, buf1, tb0, tb1, gs0, gs1, os0, os1):
        wid = lax.axis_index("s") * _NC + lax.axis_index("c")
        gsems = (gs0, gs1)
        osems = (os0, os1)

        def run_pass(unit_of_it, nit, v0_of, vw, bufs):
            def coords(it):
                tr, cv = unit_of_it(it)
                f = tr // 8
                d0 = pl.multiple_of((tr % 8) * 8, 8)
                v0 = v0_of(cv)
                return f, d0, v0, tr

            def issue(it, b):
                f, d0, v0, _ = coords(it)
                pltpu.async_copy(
                    tab_hbm.at[f, pl.ds(d0, 8), pl.ds(v0, vw)],
                    bufs[b].at[:, pl.ds(0, vw)], gsems[b])

            def wait_in(it, b):
                f, d0, v0, _ = coords(it)
                pltpu.make_async_copy(
                    tab_hbm.at[f, pl.ds(d0, 8), pl.ds(v0, vw)],
                    bufs[b].at[:, pl.ds(0, vw)], gsems[b]).wait()

            def drain_out(b):
                for dd in range(8):
                    pltpu.make_async_copy(
                        bufs[b].at[dd, pl.ds(0, vw)],
                        out_hbm.at[pl.ds(0, vw)], osems[b]).wait()

            issue(0, 0)

            def step(i, _):
                for b in range(2):
                    it = 2 * i + b
                    nb = (b + 1) % 2

                    @pl.when(it + 1 < nit)
                    def _():
                        issue(it + 1, nb)

                    f, d0, v0, tr = coords(it)
                    wait_in(it, b)

                    @pl.when(it >= 2)
                    def _():
                        drain_out(b)

                    rbase = ((tr // 8) * _D + (tr % 8) * 8) * _VP + v0
                    for dd in range(8):
                        pltpu.async_copy(
                            bufs[b].at[dd, pl.ds(0, vw)],
                            out_hbm.at[pl.ds(
                                pl.multiple_of(rbase + dd * _VP, 8), vw)],
                            osems[b])
                return 0

            lax.fori_loop(0, nit // 2, step, 0)
            for b in range(2):
                drain_out(b)

        run_pass(
            lambda it: ((wid * (4160 // _NW) + it) // _K0_NVW,
                        (wid * (4160 // _NW) + it) % _K0_NVW),
            4160 // _NW,
            lambda cv: cv * _K0_VW,
            _K0_VW, (buf0, buf1))
        tail_units = lambda it: (jnp.minimum(it * _NW + wid, _K0_TR - 1), 0)
        run_pass(tail_units, 8, lambda cv: _K0_TAIL0, _K0_TAILW, (buf0, buf1))
        run_pass(tail_units, 8, lambda cv: _K0_TB0, _K0_TBW, (tb0, tb1))

    return k(tabT)


def _sc_gather(tab1, base_idx):
    mesh = plsc.VectorSubcoreMesh(core_axis_name="c", subcore_axis_name="s")

    @functools.partial(
        pl.kernel,
        mesh=mesh,
        out_type=jax.ShapeDtypeStruct((_HROWS * _D,), jnp.float32),
        scratch_types=[
            pltpu.VMEM((_HRPW,), jnp.int32),
            pltpu.VMEM((_CR * _D,), jnp.int32),
            pltpu.VMEM((_CR * _D,), jnp.int32),
            pltpu.VMEM((_CR * _D,), jnp.float32),
            pltpu.VMEM((_CR * _D,), jnp.float32),
            pltpu.SemaphoreType.DMA,
            pltpu.SemaphoreType.DMA,
            pltpu.SemaphoreType.DMA,
            pltpu.SemaphoreType.DMA,
        ],
        compiler_params=pltpu.CompilerParams(
            use_tc_tiling_on_sc=False, needs_layout_passes=False),
    )
    def k(tab_hbm, base_hbm, out_hbm, base_v, ei0, ei1, buf0, buf1,
          gs0, gs1, os0, os1):
        wid = lax.axis_index("s") * _NC + lax.axis_index("c")
        rbase = wid * _HRPW
        pltpu.sync_copy(base_hbm.at[pl.ds(rbase, _HRPW)], base_v)
        eis = (ei0, ei1)
        bufs = (buf0, buf1)
        gsems = (gs0, gs1)
        osems = (os0, os1)

        def gen_idx(c, ei):
            for g in range(_CR // 16):
                j16 = lax.iota(jnp.int32, 16) + (16 * g)
                b16 = base_v[pl.ds(c * _CR + 16 * g, 16)]

                def body_d(d4, _):
                    for u in range(4):
                        d = d4 * 4 + u
                        plsc.store_scatter(
                            ei, [j16 * _D + d], b16 + d * _VP)
                    return 0

                lax.fori_loop(0, _D // 4, body_d, 0)

        def issue(c, b):
            pltpu.async_copy(tab_hbm.at[eis[b]], bufs[b], gsems[b])

        gen_idx(0, ei0)
        issue(0, 0)

        def step(i, _):
            for b in range(2):
                c = 2 * i + b
                nb = (b + 1) % 2

                @pl.when(c + 1 < _NCH)
                def _():
                    gen_idx(c + 1, eis[nb])
                    issue(c + 1, nb)

                pltpu.make_async_copy(
                    tab_hbm.at[eis[b]], bufs[b], gsems[b]).wait()

                @pl.when(c >= 2)
                def _():
                    pltpu.make_async_copy(
                        bufs[b], out_hbm.at[pl.ds(0, _CR * _D)],
                        osems[b]).wait()

                pltpu.async_copy(
                    bufs[b],
                    out_hbm.at[pl.ds((rbase + c * _CR) * _D, _CR * _D)],
                    osems[b])
            return 0

        lax.fori_loop(0, _NCH // 2, step, 0)
        for b in range(2):
            pltpu.make_async_copy(
                bufs[b], out_hbm.at[pl.ds(0, _CR * _D)], osems[b]).wait()

    return k(tab1, base_idx)


def _tc_body(dfp, sx, dW0, db0, dW1, db1, dW2, db2,
             cW0, cb0, cW1, cb1, cW2, cb2,
             tW0d, tW0x, tb0, tW1, tb1, tW2, tb2, tW3, tb3,
             fW, fb, out):
    f32 = jnp.float32
    bf = jnp.bfloat16
    mm = lambda a, w: jnp.dot(a.astype(bf), w[...],
                              preferred_element_type=f32)
    x = jnp.maximum(mm(dfp[...], dW0) + db0[...], 0.0)
    x = jnp.maximum(mm(x, dW1) + db1[...], 0.0)
    dxp = jnp.maximum(mm(x, dW2) + db2[...], 0.0)
    x0 = jnp.concatenate([dxp, sx[...]], axis=1)
    xi = x0
    for cW, cb in ((cW0, cb0), (cW1, cb1), (cW2, cb2)):
        xi = x0 * (mm(xi, cW) + cb[...]) + xi
    h = mm(dxp, tW0d) + mm(xi, tW0x) + tb0[...]
    h = jnp.maximum(h, 0.0)
    h = jnp.maximum(mm(h, tW1) + tb1[...], 0.0)
    h = jnp.maximum(mm(h, tW2) + tb2[...], 0.0)
    h = jnp.maximum(mm(h, tW3) + tb3[...], 0.0)
    z = mm(h, fW) + fb[...]
    out[...] = 1.0 / (1.0 + jnp.exp(-z))


def _dense_stack(dfp, sx, weights):
    nbatch = dfp.shape[0]
    nb = nbatch // _BB
    full = lambda s: pl.BlockSpec(s, lambda i: (0, 0))
    in_specs = [
        pl.BlockSpec((_BB, 128), lambda i: (i, 0)),
        pl.BlockSpec((_BB, _F * _D), lambda i: (i, 0)),
    ] + [full(w.shape) for w in weights]
    return pl.pallas_call(
        _tc_body,
        grid=(nb,),
        in_specs=in_specs,
        out_specs=pl.BlockSpec((_BB, 1), lambda i: (i, 0)),
        out_shape=jax.ShapeDtypeStruct((nbatch, 1), jnp.float32),
        compiler_params=pltpu.CompilerParams(
            dimension_semantics=("arbitrary",),
            vmem_limit_bytes=63 * 1024 * 1024,
        ),
    )(dfp, sx, *weights)


def kernel(dense_features, sparse_features, tables, dense_Ws, dense_bs,
           cross_Ws, cross_bs, top_Ws, top_bs, final_W, final_b):
    f32 = jnp.float32
    tabT = jnp.transpose(tables, (0, 2, 1))
    tab1 = _sc_depad(tabT)
    base_idx = (sparse_features.astype(jnp.int32)
                + (jnp.arange(_F, dtype=jnp.int32) * (_D * _VP))[None, :]
                ).reshape(-1)
    sxs = [
        _sc_gather(tab1, base_idx[h * _HROWS:(h + 1) * _HROWS]
                   ).reshape(_B // 2, _F * _D)
        for h in range(2)
    ]

    z = lambda r, c: jnp.zeros((r, c), f32)
    dfp = jnp.pad(dense_features, ((0, 0), (0, 128 - dense_features.shape[1])))
    dW0 = jnp.pad(dense_Ws[0], ((0, 128 - dense_Ws[0].shape[0]), (0, 0)))
    db0 = dense_bs[0].reshape(1, -1)
    dW1 = dense_Ws[1]
    db1 = dense_bs[1].reshape(1, -1)
    dW2 = jnp.pad(dense_Ws[2], ((0, 0), (0, 64)))
    db2 = jnp.pad(dense_bs[2], ((0, 64),)).reshape(1, 128)

    cWs, cbs = [], []
    for W, b in zip(cross_Ws, cross_bs):
        t = jnp.concatenate([W[:64], z(64, W.shape[1]), W[64:]], axis=0)
        Wp = jnp.concatenate([t[:, :64], z(t.shape[0], 64), t[:, 64:]], axis=1)
        cWs.append(Wp)
        cbs.append(jnp.concatenate(
            [b[:64], jnp.zeros((64,), f32), b[64:]]).reshape(1, -1))

    tW0d = jnp.concatenate([top_Ws[0][:64], z(64, 1024)], axis=0)
    tW0x = jnp.concatenate(
        [top_Ws[0][64:128], z(64, 1024), top_Ws[0][128:]], axis=0)
    tb0 = top_bs[0].reshape(1, -1)
    tW1, tb1 = top_Ws[1], top_bs[1].reshape(1, -1)
    tW2, tb2 = top_Ws[2], top_bs[2].reshape(1, -1)
    tW3, tb3 = top_Ws[3], top_bs[3].reshape(1, -1)
    fb = final_b.reshape(1, 1)

    bf = jnp.bfloat16
    weights = [dW0.astype(bf), db0, dW1.astype(bf), db1, dW2.astype(bf), db2,
               cWs[0].astype(bf), cbs[0], cWs[1].astype(bf), cbs[1],
               cWs[2].astype(bf), cbs[2],
               tW0d.astype(bf), tW0x.astype(bf), tb0,
               tW1.astype(bf), tb1, tW2.astype(bf), tb2, tW3.astype(bf), tb3,
               final_W.astype(bf), fb]
    outs = [
        _dense_stack(dfp[h * (_B // 2):(h + 1) * (_B // 2)], sxs[h], weights)
        for h in range(2)
    ]
    return jnp.concatenate(outs, axis=0)


# speedup vs baseline: 1.7199x; 1.0090x over previous
import functools

import jax
import jax.numpy as jnp
from jax import lax
from jax.experimental import pallas as pl
from jax.experimental.pallas import tpu as pltpu
from jax.experimental.pallas import tpu_sc as plsc

_F = 26
_V = 100000
_VP = 100096
_D = 64
_B = 4096

_NC = 2
_NS = 16
_NW = _NC * _NS
_ROWS = _B * _F
_RPW = _ROWS // _NW

_TAB1 = _F * _D * _VP

_K0_TR = _F * _D // 8
_K0_VW = 4864
_K0_NVW = 20
_K0_TAIL0 = _K0_NVW * _K0_VW
_K0_TAILW = 2688
_K0_TB0 = _K0_TAIL0 + _K0_TAILW
_K0_TBW = _V - _K0_TB0

_NSPL = 4
_HROWS = _ROWS // _NSPL
_HRPW = _HROWS // _NW
_CR = 32
_NCH = _HRPW // _CR

_BB = 512


def _sc_depad(tabT):
    mesh = plsc.VectorSubcoreMesh(core_axis_name="c", subcore_axis_name="s")

    @functools.partial(
        pl.kernel,
        mesh=mesh,
        out_type=jax.ShapeDtypeStruct((_TAB1,), jnp.float32),
        scratch_types=[
            pltpu.VMEM((8, _K0_VW), jnp.float32),
            pltpu.VMEM((8, _K0_VW), jnp.float32),
            pltpu.VMEM((8, _K0_TBW), jnp.float32),
            pltpu.VMEM((8, _K0_TBW), jnp.float32),
            pltpu.SemaphoreType.DMA,
            pltpu.SemaphoreType.DMA,
            pltpu.SemaphoreType.DMA,
            pltpu.SemaphoreType.DMA,
        ],
        compiler_params=pltpu.CompilerParams(
            use_tc_tiling_on_sc=True, needs_layout_passes=False),
    )
    def k(tab_hbm, out_hbm, buf0, buf1, tb0, tb1, gs0, gs1, os0, os1):
        wid = lax.axis_index("s") * _NC + lax.axis_index("c")
        gsems = (gs0, gs1)
        osems = (os0, os1)

        def run_pass(unit_of_it, nit, v0_of, vw, bufs):
            def coords(it):
                tr, cv = unit_of_it(it)
                f = tr // 8
                d0 = pl.multiple_of((tr % 8) * 8, 8)
                v0 = v0_of(cv)
                return f, d0, v0, tr

            def issue(it, b):
                f, d0, v0, _ = coords(it)
                pltpu.async_copy(
                    tab_hbm.at[f, pl.ds(d0, 8), pl.ds(v0, vw)],
                    bufs[b].at[:, pl.ds(0, vw)], gsems[b])

            def wait_in(it, b):
                f, d0, v0, _ = coords(it)
                pltpu.make_async_copy(
                    tab_hbm.at[f, pl.ds(d0, 8), pl.ds(v0, vw)],
                    bufs[b].at[:, pl.ds(0, vw)], gsems[b]).wait()

            def drain_out(b):
                for dd in range(8):
                    pltpu.make_async_copy(
                        bufs[b].at[dd, pl.ds(0, vw)],
                        out_hbm.at[pl.ds(0, vw)], osems[b]).wait()

            issue(0, 0)

            def step(i, _):
                for b in range(2):
                    it = 2 * i + b
                    nb = (b + 1) % 2

                    @pl.when(it + 1 < nit)
                    def _():
                        issue(it + 1, nb)

                    f, d0, v0, tr = coords(it)
                    wait_in(it, b)

                    @pl.when(it >= 2)
                    def _():
                        drain_out(b)

                    rbase = ((tr // 8) * _D + (tr % 8) * 8) * _VP + v0
                    for dd in range(8):
                        pltpu.async_copy(
                            bufs[b].at[dd, pl.ds(0, vw)],
                            out_hbm.at[pl.ds(
                                pl.multiple_of(rbase + dd * _VP, 8), vw)],
                            osems[b])
                return 0

            lax.fori_loop(0, nit // 2, step, 0)
            for b in range(2):
                drain_out(b)

        run_pass(
            lambda it: ((wid * (4160 // _NW) + it) // _K0_NVW,
                        (wid * (4160 // _NW) + it) % _K0_NVW),
            4160 // _NW,
            lambda cv: cv * _K0_VW,
            _K0_VW, (buf0, buf1))
        tail_units = lambda it: (jnp.minimum(it * _NW + wid, _K0_TR - 1), 0)
        run_pass(tail_units, 8, lambda cv: _K0_TAIL0, _K0_TAILW, (buf0, buf1))
        run_pass(tail_units, 8, lambda cv: _K0_TB0, _K0_TBW, (tb0, tb1))

    return k(tabT)


def _sc_gather(tab1, base_idx):
    mesh = plsc.VectorSubcoreMesh(core_axis_name="c", subcore_axis_name="s")

    @functools.partial(
        pl.kernel,
        mesh=mesh,
        out_type=jax.ShapeDtypeStruct((_HROWS * _D,), jnp.float32),
        scratch_types=[
            pltpu.VMEM((_HRPW,), jnp.int32),
            pltpu.VMEM((_CR * _D,), jnp.int32),
            pltpu.VMEM((_CR * _D,), jnp.int32),
            pltpu.VMEM((_CR * _D,), jnp.float32),
            pltpu.VMEM((_CR * _D,), jnp.float32),
            pltpu.SemaphoreType.DMA,
            pltpu.SemaphoreType.DMA,
            pltpu.SemaphoreType.DMA,
            pltpu.SemaphoreType.DMA,
        ],
        compiler_params=pltpu.CompilerParams(
            use_tc_tiling_on_sc=False, needs_layout_passes=False),
    )
    def k(tab_hbm, base_hbm, out_hbm, base_v, ei0, ei1, buf0, buf1,
          gs0, gs1, os0, os1):
        wid = lax.axis_index("s") * _NC + lax.axis_index("c")
        rbase = wid * _HRPW
        pltpu.sync_copy(base_hbm.at[pl.ds(rbase, _HRPW)], base_v)
        eis = (ei0, ei1)
        bufs = (buf0, buf1)
        gsems = (gs0, gs1)
        osems = (os0, os1)

        def gen_idx(c, ei):
            for g in range(_CR // 16):
                j16 = lax.iota(jnp.int32, 16) + (16 * g)
                b16 = base_v[pl.ds(c * _CR + 16 * g, 16)]

                def body_d(d4, _):
                    for u in range(4):
                        d = d4 * 4 + u
                        plsc.store_scatter(
                            ei, [j16 * _D + d], b16 + d * _VP)
                    return 0

                lax.fori_loop(0, _D // 4, body_d, 0)

        def issue(c, b):
            pltpu.async_copy(tab_hbm.at[eis[b]], bufs[b], gsems[b])

        gen_idx(0, ei0)
        issue(0, 0)

        def step(i, _):
            for b in range(2):
                c = 2 * i + b
                nb = (b + 1) % 2

                @pl.when(c + 1 < _NCH)
                def _():
                    gen_idx(c + 1, eis[nb])
                    issue(c + 1, nb)

                pltpu.make_async_copy(
                    tab_hbm.at[eis[b]], bufs[b], gsems[b]).wait()

                @pl.when(c >= 2)
                def _():
                    pltpu.make_async_copy(
                        bufs[b], out_hbm.at[pl.ds(0, _CR * _D)],
                        osems[b]).wait()

                pltpu.async_copy(
                    bufs[b],
                    out_hbm.at[pl.ds((rbase + c * _CR) * _D, _CR * _D)],
                    osems[b])
            return 0

        lax.fori_loop(0, _NCH // 2, step, 0)
        for b in range(2):
            pltpu.make_async_copy(
                bufs[b], out_hbm.at[pl.ds(0, _CR * _D)], osems[b]).wait()

    return k(tab1, base_idx)


def _tc_body(dfp, sx, dW0, db0, dW1, db1, dW2, db2,
             cW0, cb0, cW1, cb1, cW2, cb2,
             tW0d, tW0x, tb0, tW1, tb1, tW2, tb2, tW3, tb3,
             fW, fb, out):
    f32 = jnp.float32
    bf = jnp.bfloat16
    mm = lambda a, w: jnp.dot(a.astype(bf), w[...],
                              preferred_element_type=f32)
    x = jnp.maximum(mm(dfp[...], dW0) + db0[...], 0.0)
    x = jnp.maximum(mm(x, dW1) + db1[...], 0.0)
    dxp = jnp.maximum(mm(x, dW2) + db2[...], 0.0)
    x0 = jnp.concatenate([dxp, sx[...]], axis=1)
    xi = x0
    for cW, cb in ((cW0, cb0), (cW1, cb1), (cW2, cb2)):
        xi = x0 * (mm(xi, cW) + cb[...]) + xi
    h = mm(dxp, tW0d) + mm(xi, tW0x) + tb0[...]
    h = jnp.maximum(h, 0.0)
    h = jnp.maximum(mm(h, tW1) + tb1[...], 0.0)
    h = jnp.maximum(mm(h, tW2) + tb2[...], 0.0)
    h = jnp.maximum(mm(h, tW3) + tb3[...], 0.0)
    z = mm(h, fW) + fb[...]
    out[...] = 1.0 / (1.0 + jnp.exp(-z))


def _dense_stack(dfp, sx, weights):
    nbatch = dfp.shape[0]
    nb = nbatch // _BB
    full = lambda s: pl.BlockSpec(s, lambda i: (0, 0))
    in_specs = [
        pl.BlockSpec((_BB, 128), lambda i: (i, 0)),
        pl.BlockSpec((_BB, _F * _D), lambda i: (i, 0)),
    ] + [full(w.shape) for w in weights]
    return pl.pallas_call(
        _tc_body,
        grid=(nb,),
        in_specs=in_specs,
        out_specs=pl.BlockSpec((_BB, 1), lambda i: (i, 0)),
        out_shape=jax.ShapeDtypeStruct((nbatch, 1), jnp.float32),
        compiler_params=pltpu.CompilerParams(
            dimension_semantics=("arbitrary",),
            vmem_limit_bytes=63 * 1024 * 1024,
        ),
    )(dfp, sx, *weights)


def kernel(dense_features, sparse_features, tables, dense_Ws, dense_bs,
           cross_Ws, cross_bs, top_Ws, top_bs, final_W, final_b):
    f32 = jnp.float32
    tabT = jnp.transpose(tables, (0, 2, 1))
    tab1 = _sc_depad(tabT)
    base_idx = (sparse_features.astype(jnp.int32)
                + (jnp.arange(_F, dtype=jnp.int32) * (_D * _VP))[None, :]
                ).reshape(-1)
    sxs = [
        _sc_gather(tab1, base_idx[h * _HROWS:(h + 1) * _HROWS]
                   ).reshape(_B // _NSPL, _F * _D)
        for h in range(_NSPL)
    ]

    z = lambda r, c: jnp.zeros((r, c), f32)
    dfp = jnp.pad(dense_features, ((0, 0), (0, 128 - dense_features.shape[1])))
    dW0 = jnp.pad(dense_Ws[0], ((0, 128 - dense_Ws[0].shape[0]), (0, 0)))
    db0 = dense_bs[0].reshape(1, -1)
    dW1 = dense_Ws[1]
    db1 = dense_bs[1].reshape(1, -1)
    dW2 = jnp.pad(dense_Ws[2], ((0, 0), (0, 64)))
    db2 = jnp.pad(dense_bs[2], ((0, 64),)).reshape(1, 128)

    cWs, cbs = [], []
    for W, b in zip(cross_Ws, cross_bs):
        t = jnp.concatenate([W[:64], z(64, W.shape[1]), W[64:]], axis=0)
        Wp = jnp.concatenate([t[:, :64], z(t.shape[0], 64), t[:, 64:]], axis=1)
        cWs.append(Wp)
        cbs.append(jnp.concatenate(
            [b[:64], jnp.zeros((64,), f32), b[64:]]).reshape(1, -1))

    tW0d = jnp.concatenate([top_Ws[0][:64], z(64, 1024)], axis=0)
    tW0x = jnp.concatenate(
        [top_Ws[0][64:128], z(64, 1024), top_Ws[0][128:]], axis=0)
    tb0 = top_bs[0].reshape(1, -1)
    tW1, tb1 = top_Ws[1], top_bs[1].reshape(1, -1)
    tW2, tb2 = top_Ws[2], top_bs[2].reshape(1, -1)
    tW3, tb3 = top_Ws[3], top_bs[3].reshape(1, -1)
    fb = final_b.reshape(1, 1)

    bf = jnp.bfloat16
    weights = [dW0.astype(bf), db0, dW1.astype(bf), db1, dW2.astype(bf), db2,
               cWs[0].astype(bf), cbs[0], cWs[1].astype(bf), cbs[1],
               cWs[2].astype(bf), cbs[2],
               tW0d.astype(bf), tW0x.astype(bf), tb0,
               tW1.astype(bf), tb1, tW2.astype(bf), tb2, tW3.astype(bf), tb3,
               final_W.astype(bf), fb]
    hb = _B // _NSPL
    outs = [
        _dense_stack(dfp[h * hb:(h + 1) * hb], sxs[h], weights)
        for h in range(_NSPL)
    ]
    return jnp.concatenate(outs, axis=0)
